# baseline (speedup 1.0000x reference)
import functools

import jax
import jax.numpy as jnp
from jax import lax
from jax.experimental import pallas as pl
from jax.experimental.pallas import tpu as pltpu
from jax.experimental.pallas import tpu_sc as plsc

BEAM = 3
TOPK = 3
ROW = 16 * 2048 * 64
NC = 2
NS = 16
NW = NC * NS
CH = ROW // NW

_NEG = float("-inf")
_BIG = 2**30


def _topk_body(logits_ref, save_id_ref, prev_ref, tokens_ref, save_out_ref,
               prob_ref, maxidx_ref, bsel_ref):
    x = logits_ref[...]
    col = lax.broadcasted_iota(jnp.int32, x.shape, 1)

    m = jnp.max(x, axis=1, keepdims=True)
    s = jnp.sum(jnp.exp(x - m), axis=1, keepdims=True)
    lse = m + jnp.log(s)

    work = x
    vals, idxs = [], []
    for _ in range(TOPK):
        v = jnp.max(work, axis=1, keepdims=True)
        i = jnp.min(jnp.where(work == v, col, _BIG), axis=1,
                    keepdims=True)
        vals.append(v)
        idxs.append(i)
        work = jnp.where(col == i, _NEG, work)
    topv = jnp.concatenate(vals, axis=1)
    topi = jnp.concatenate(idxs, axis=1)

    cur = topv - lse + prev_ref[...]

    rowk = lax.broadcasted_iota(jnp.int32, cur.shape, 0)
    colk = lax.broadcasted_iota(jnp.int32, cur.shape, 1)
    fi = rowk * TOPK + colk

    tok_sc, bsel_sc, prob_sc = [], [], []
    workc = cur
    for _ in range(BEAM):
        v = jnp.max(workc)
        f = jnp.min(jnp.where(workc == v, fi, _BIG))
        tok = jnp.sum(jnp.where(fi == f, topi, 0))
        prob_sc.append(v)
        bsel_sc.append(jnp.minimum(f // TOPK, BEAM - 1))
        tok_sc.append(tok)
        workc = jnp.where(fi == f, _NEG, workc)

    row31 = lax.broadcasted_iota(jnp.int32, (BEAM, 1), 0)
    tokens = jnp.zeros((BEAM, 1), jnp.int32)
    probs = jnp.zeros((BEAM, 1), jnp.float32)
    for j in range(BEAM):
        tokens = jnp.where(row31 == j, tok_sc[j], tokens)
        probs = jnp.where(row31 == j, prob_sc[j], probs)
    tokens_ref[...] = tokens
    prob_ref[...] = probs
    maxidx_ref[...] = jnp.full((1, 1), tok_sc[0], jnp.int32)

    sid = save_id_ref[...]
    rowm = lax.broadcasted_iota(jnp.int32, sid.shape, 0)
    g = jnp.zeros(sid.shape, jnp.int32)
    for j in range(BEAM):
        rowj = jnp.sum(jnp.where(rowm == bsel_sc[j], sid, 0), axis=0,
                       keepdims=True)
        g = jnp.where(rowm == j, rowj, g)
    save_out_ref[:, :128] = g
    save_out_ref[:, 128:129] = tokens

    col16 = lax.broadcasted_iota(jnp.int32, (1, 16), 1)
    bsel = jnp.zeros((1, 16), jnp.int32)
    for j in range(BEAM):
        bsel = jnp.where(col16 == j, bsel_sc[j], bsel)
    bsel_ref[...] = bsel


_NBUF = 2
_CCOLS = 256
_NSUB = 1024 // _CCOLS
_NSC = 12
_NGRP = _NSC * _NSUB


def _sc_gather_body(bsel_hbm, *rest):
    kvs = rest[:_NSC]
    outs = rest[_NSC:2 * _NSC]
    bi_vmem = rest[2 * _NSC]
    gbufs = rest[2 * _NSC + 1:2 * _NSC + 1 + _NBUF]
    isems = rest[2 * _NSC + 1 + _NBUF:2 * _NSC + 1 + 2 * _NBUF]
    osems = rest[2 * _NSC + 1 + 2 * _NBUF:2 * _NSC + 1 + 3 * _NBUF]

    wid = lax.axis_index("s") * NC + lax.axis_index("c")
    head = wid // 2
    half = wid % 2

    pltpu.sync_copy(bsel_hbm, bi_vmem)
    v = bi_vmem[...]
    lane = lax.iota(jnp.int32, 16)
    srcs = [jnp.max(jnp.where(lane == b, v, 0), axis=0) for b in range(BEAM)]
    need = [
        (srcs[0] == c) | (srcs[1] == c) | (srcs[2] == c) for c in range(BEAM)
    ]

    def _addr(g):
        i, p = divmod(g, _NSUB)
        return i, half * 1024 + p * _CCOLS

    def in_descs(g):
        i, col0 = _addr(g)
        r = g % _NBUF
        return [
            pltpu.make_async_copy(
                kvs[i].at[c, head, :, pl.ds(col0, _CCOLS)],
                gbufs[r].at[c], isems[r])
            for c in range(BEAM)
        ]

    def out_descs(g):
        i, col0 = _addr(g)
        r = g % _NBUF
        return [
            pltpu.make_async_copy(
                gbufs[r].at[srcs[b]],
                outs[i].at[b, head, :, pl.ds(col0, _CCOLS)], osems[r])
            for b in range(BEAM)
        ]

    lead = _NBUF - 2

    def start_ins(g):
        ins = in_descs(g)
        for c in range(BEAM):
            pl.when(need[c])(ins[c].start)
        return ins

    in_h = {}
    out_h = {}
    for g in range(lead):
        in_h[g] = start_ins(g)
    for g in range(_NGRP):
        h = g + lead
        if h < _NGRP:
            if h >= _NBUF:
                for d in out_h[h - _NBUF]:
                    d.wait()
            in_h[h] = start_ins(h)
        for c in range(BEAM):
            pl.when(need[c])(in_h[g][c].wait)
        out_h[g] = out_descs(g)
        for d in out_h[g]:
            d.start()
    for g in range(max(0, _NGRP - _NBUF), _NGRP):
        for d in out_h[g]:
            d.wait()


@functools.lru_cache(maxsize=1)
def _make_sc_gather():
    kv4t = jax.ShapeDtypeStruct((BEAM, 16, 64, 2048), jnp.float32)
    return pl.kernel(
        _sc_gather_body,
        out_type=[kv4t] * _NSC,
        mesh=plsc.VectorSubcoreMesh(core_axis_name="c", subcore_axis_name="s"),
        scratch_types=(
            [pltpu.VMEM((16,), jnp.int32)]
            + [pltpu.VMEM((BEAM, 64, _CCOLS), jnp.float32)
               for _ in range(_NBUF)]
            + [pltpu.SemaphoreType.DMA for _ in range(2 * _NBUF)]
        ),
        compiler_params=pltpu.CompilerParams(needs_layout_passes=False),
    )


def kernel(kv_0, kv_1, kv_2, kv_3, kv_4, kv_5, kv_6, kv_7, kv_8, kv_9,
           kv_10, kv_11, logits, save_id, previous_prob, beam_size, topK):
    kvs = [kv_0, kv_1, kv_2, kv_3, kv_4, kv_5, kv_6, kv_7, kv_8, kv_9,
           kv_10, kv_11]

    tokens, save_out, probs, maxidx, bsel = pl.pallas_call(
        _topk_body,
        out_shape=[
            jax.ShapeDtypeStruct((BEAM, 1), jnp.int32),
            jax.ShapeDtypeStruct((BEAM, 129), jnp.int32),
            jax.ShapeDtypeStruct((BEAM, 1), jnp.float32),
            jax.ShapeDtypeStruct((1, 1), jnp.int32),
            jax.ShapeDtypeStruct((1, 16), jnp.int32),
        ],
    )(logits, save_id, previous_prob)

    kvt = [jnp.transpose(kv, (0, 1, 3, 2)) for kv in kvs]
    outs_sc = _make_sc_gather()(bsel.reshape(16), *kvt)
    save_kvs = [jnp.transpose(o, (0, 1, 3, 2)) for o in outs_sc]

    return (*save_kvs, tokens, save_out, probs, maxidx.reshape(1))


# speedup vs baseline: 1.0109x; 1.0109x over previous
import functools

import jax
import jax.numpy as jnp
from jax import lax
from jax.experimental import pallas as pl
from jax.experimental.pallas import tpu as pltpu
from jax.experimental.pallas import tpu_sc as plsc

BEAM = 3
TOPK = 3
NC = 2

_NEG = float("-inf")
_BIG = 2**30


def _topk_body(logits_ref, save_id_ref, prev_ref, tokens_ref, save_out_ref,
               prob_ref, maxidx_ref, bsel_ref):
    x = logits_ref[...]
    col = lax.broadcasted_iota(jnp.int32, x.shape, 1)

    m = jnp.max(x, axis=1, keepdims=True)
    s = jnp.sum(jnp.exp(x - m), axis=1, keepdims=True)
    lse = m + jnp.log(s)

    work = x
    vals, idxs = [], []
    for _ in range(TOPK):
        v = jnp.max(work, axis=1, keepdims=True)
        i = jnp.min(jnp.where(work == v, col, _BIG), axis=1,
                    keepdims=True)
        vals.append(v)
        idxs.append(i)
        work = jnp.where(col == i, _NEG, work)
    topv = jnp.concatenate(vals, axis=1)
    topi = jnp.concatenate(idxs, axis=1)

    cur = topv - lse + prev_ref[...]

    rowk = lax.broadcasted_iota(jnp.int32, cur.shape, 0)
    colk = lax.broadcasted_iota(jnp.int32, cur.shape, 1)
    fi = rowk * TOPK + colk

    tok_sc, bsel_sc, prob_sc = [], [], []
    workc = cur
    for _ in range(BEAM):
        v = jnp.max(workc)
        f = jnp.min(jnp.where(workc == v, fi, _BIG))
        tok = jnp.sum(jnp.where(fi == f, topi, 0))
        prob_sc.append(v)
        bsel_sc.append(jnp.minimum(f // TOPK, BEAM - 1))
        tok_sc.append(tok)
        workc = jnp.where(fi == f, _NEG, workc)

    row31 = lax.broadcasted_iota(jnp.int32, (BEAM, 1), 0)
    tokens = jnp.zeros((BEAM, 1), jnp.int32)
    probs = jnp.zeros((BEAM, 1), jnp.float32)
    for j in range(BEAM):
        tokens = jnp.where(row31 == j, tok_sc[j], tokens)
        probs = jnp.where(row31 == j, prob_sc[j], probs)
    tokens_ref[...] = tokens
    prob_ref[...] = probs
    maxidx_ref[...] = jnp.full((1, 1), tok_sc[0], jnp.int32)

    sid = save_id_ref[...]
    rowm = lax.broadcasted_iota(jnp.int32, sid.shape, 0)
    g = jnp.zeros(sid.shape, jnp.int32)
    for j in range(BEAM):
        rowj = jnp.sum(jnp.where(rowm == bsel_sc[j], sid, 0), axis=0,
                       keepdims=True)
        g = jnp.where(rowm == j, rowj, g)
    save_out_ref[:, :128] = g
    save_out_ref[:, 128:129] = tokens

    col16 = lax.broadcasted_iota(jnp.int32, (1, 16), 1)
    bsel = jnp.zeros((1, 16), jnp.int32)
    for j in range(BEAM):
        bsel = jnp.where(col16 == j, bsel_sc[j], bsel)
    bsel_ref[...] = bsel


_NBUF = 2
_CCOLS = 256
_NSUB = 1024 // _CCOLS
_NSC = 12
_NGRP = _NSC * _NSUB


def _sc_gather_body(bsel_hbm, *rest):
    kvs = rest[:_NSC]
    outs = rest[_NSC:2 * _NSC]
    bi_vmem = rest[2 * _NSC]
    gbufs = rest[2 * _NSC + 1:2 * _NSC + 1 + _NBUF]
    isems = rest[2 * _NSC + 1 + _NBUF:2 * _NSC + 1 + 2 * _NBUF]
    osems = rest[2 * _NSC + 1 + 2 * _NBUF:2 * _NSC + 1 + 3 * _NBUF]

    wid = lax.axis_index("s") * NC + lax.axis_index("c")
    head = wid // 2
    half = wid % 2

    pltpu.sync_copy(bsel_hbm, bi_vmem)
    v = bi_vmem[...]
    lane = lax.iota(jnp.int32, 16)
    srcs = [jnp.max(jnp.where(lane == b, v, 0), axis=0) for b in range(BEAM)]
    need = [
        (srcs[0] == c) | (srcs[1] == c) | (srcs[2] == c) for c in range(BEAM)
    ]

    def _addr(g):
        i, p = divmod(g, _NSUB)
        return i, half * 1024 + p * _CCOLS

    def in_descs(g):
        i, col0 = _addr(g)
        r = g % _NBUF
        return [
            pltpu.make_async_copy(
                kvs[i].at[c, head, :, pl.ds(col0, _CCOLS)],
                gbufs[r].at[c], isems[r])
            for c in range(BEAM)
        ]

    def out_descs(g):
        i, col0 = _addr(g)
        r = g % _NBUF
        return [
            pltpu.make_async_copy(
                gbufs[r].at[srcs[b]],
                outs[i].at[b, head, :, pl.ds(col0, _CCOLS)], osems[r])
            for b in range(BEAM)
        ]

    lead = _NBUF - 2

    def start_ins(g):
        ins = in_descs(g)
        for c in range(BEAM):
            pl.when(need[c])(ins[c].start)
        return ins

    in_h = {}
    out_h = {}
    for g in range(lead):
        in_h[g] = start_ins(g)
    for g in range(_NGRP):
        h = g + lead
        if h < _NGRP:
            if h >= _NBUF:
                for d in out_h[h - _NBUF]:
                    d.wait()
            in_h[h] = start_ins(h)
        for c in range(BEAM):
            pl.when(need[c])(in_h[g][c].wait)
        out_h[g] = out_descs(g)
        for d in out_h[g]:
            d.start()
    for g in range(max(0, _NGRP - _NBUF), _NGRP):
        for d in out_h[g]:
            d.wait()


@functools.lru_cache(maxsize=1)
def _make_sc_gather():
    kv4t = jax.ShapeDtypeStruct((BEAM, 16, 64, 2048), jnp.float32)
    return pl.kernel(
        _sc_gather_body,
        out_type=[kv4t] * _NSC,
        mesh=plsc.VectorSubcoreMesh(core_axis_name="c", subcore_axis_name="s"),
        scratch_types=(
            [pltpu.VMEM((16,), jnp.int32)]
            + [pltpu.VMEM((BEAM, 64, _CCOLS), jnp.float32)
               for _ in range(_NBUF)]
            + [pltpu.SemaphoreType.DMA for _ in range(2 * _NBUF)]
        ),
        compiler_params=pltpu.CompilerParams(needs_layout_passes=False),
    )


def kernel(kv_0, kv_1, kv_2, kv_3, kv_4, kv_5, kv_6, kv_7, kv_8, kv_9,
           kv_10, kv_11, logits, save_id, previous_prob, beam_size, topK):
    kvs = [kv_0, kv_1, kv_2, kv_3, kv_4, kv_5, kv_6, kv_7, kv_8, kv_9,
           kv_10, kv_11]

    tokens, save_out, probs, maxidx, bsel = pl.pallas_call(
        _topk_body,
        out_shape=[
            jax.ShapeDtypeStruct((BEAM, 1), jnp.int32),
            jax.ShapeDtypeStruct((BEAM, 129), jnp.int32),
            jax.ShapeDtypeStruct((BEAM, 1), jnp.float32),
            jax.ShapeDtypeStruct((1, 1), jnp.int32),
            jax.ShapeDtypeStruct((1, 16), jnp.int32),
        ],
    )(logits, save_id, previous_prob)

    kvt = [jnp.transpose(kv, (0, 1, 3, 2)) for kv in kvs]
    outs_sc = _make_sc_gather()(bsel.reshape(16), *kvt)
    save_kvs = [jnp.transpose(o, (0, 1, 3, 2)) for o in outs_sc]

    return (*save_kvs, tokens, save_out, probs, maxidx.reshape(1))
